# Initial kernel scaffold; baseline (speedup 1.0000x reference)
import jax
import jax.numpy as jnp
from jax.experimental import pallas as pl


def kernel(query, key, value, proj_dir):
    raise NotImplementedError("write your pallas kernel here")


# speedup vs baseline: 7.9125x; 7.9125x over previous
import functools
import math

import numpy as np
import jax
import jax.numpy as jnp
from jax import lax
from jax.experimental import pallas as pl
from jax.experimental.pallas import tpu as pltpu
from jax.experimental.pallas import tpu_sc as plsc

LSH_NUM_PROJS = 7
NUM_BUCKETS = 1 << LSH_NUM_PROJS
BLOCK_SIZE = 256
SAMPLE_SIZE = 256

SC_CORES = 2
SC_SUBCORES = 16
SC_WORKERS = SC_CORES * SC_SUBCORES
CHUNK = 128


def _unit_hamming_distance_array(size_n):
    a = np.array([0, 1], dtype=np.int32)
    for _ in range(size_n - 1):
        a = np.concatenate([a, np.flip(a) + a.shape[0]])
    return a


_PERM_NP = _unit_hamming_distance_array(LSH_NUM_PROJS).astype(np.int32)


@functools.lru_cache(maxsize=None)
def _sampled_set_np(b, h, n_key):
    with jax.ensure_compile_time_eval():
        s = jax.random.randint(jax.random.key(1234), (b, h, SAMPLE_SIZE), 0,
                               n_key)
        return np.asarray(jax.device_get(s)).astype(np.int32)


def _hash_pos_body(q_ref, k_ref, w_ref, perm_ref, posq_ref, posk_ref, *, seq, dim):
    bh = pl.program_id(0)
    w = w_ref[...]
    perm_row = perm_ref[...]
    lanes = lax.broadcasted_iota(jnp.int32, (1, NUM_BUCKETS), 1)
    rr = lax.broadcasted_iota(jnp.int32, (CHUNK, CHUNK), 0)
    cc = lax.broadcasted_iota(jnp.int32, (CHUNK, CHUNK), 1)
    tril_inc = (rr >= cc).astype(jnp.float32)
    triu_strict = (rr < cc).astype(jnp.float32)
    lane8 = lax.broadcasted_iota(jnp.int32, (1, 8), 1)

    def positions(x):
        proj = jnp.dot(x, w, preferred_element_type=jnp.float32)
        bits = (proj > 0).astype(jnp.int32)
        bin_id = jnp.sum(lax.shift_left(bits, lane8), axis=1,
                         keepdims=True)
        oh_bin = (bin_id == lanes).astype(jnp.int32)
        hsh = jnp.sum(oh_bin * perm_row, axis=1, keepdims=True)
        oh = (hsh == lanes).astype(jnp.float32)

        running = jnp.zeros((1, NUM_BUCKETS), jnp.float32)
        parts = []
        for c in range(seq // CHUNK):
            blk = oh[c * CHUNK:(c + 1) * CHUNK, :]
            cum_c = (
                jax.lax.dot_general(
                    tril_inc, blk, (((1,), (0,)), ((), ())),
                    precision=lax.Precision.HIGHEST,
                    preferred_element_type=jnp.float32,
                )
                + running
            )
            parts.append(cum_c)
            running = running + jnp.sum(blk, axis=0, keepdims=True)
        cum = jnp.concatenate(parts, axis=0)
        counts = running
        offs = jax.lax.dot_general(
            counts, triu_strict, (((1,), (0,)), ((), ())),
            precision=lax.Precision.HIGHEST,
            preferred_element_type=jnp.float32,
        )
        rank_incl = jnp.sum(cum * oh, axis=1, keepdims=True)
        off_i = jnp.sum(oh * offs, axis=1, keepdims=True)
        pos = off_i + rank_incl - 1.0
        return pos.astype(jnp.int32) + bh * seq

    posq_ref[0] = positions(q_ref[0])
    posk_ref[0] = positions(k_ref[0])


def _hash_positions(q3, k3, w_pad, bh, seq, dim):
    perm = jnp.asarray(_PERM_NP.reshape(1, NUM_BUCKETS))
    out_shape = jax.ShapeDtypeStruct((bh, seq, 1), jnp.int32)
    return pl.pallas_call(
        functools.partial(_hash_pos_body, seq=seq, dim=dim),
        grid=(bh,),
        in_specs=[
            pl.BlockSpec((1, seq, dim), lambda i: (i, 0, 0)),
            pl.BlockSpec((1, seq, dim), lambda i: (i, 0, 0)),
            pl.BlockSpec((dim, 8), lambda i: (0, 0)),
            pl.BlockSpec((1, NUM_BUCKETS), lambda i: (0, 0)),
        ],
        out_specs=[
            pl.BlockSpec((1, seq, 1), lambda i: (i, 0, 0)),
            pl.BlockSpec((1, seq, 1), lambda i: (i, 0, 0)),
        ],
        out_shape=[out_shape, out_shape],
    )(q3, k3, w_pad, perm)


def _sc_sort_rows(q2, k2, v2, posq3, posk3, dim):
    n = q2.shape[0]
    rows_per_w = n // SC_WORKERS
    nch = rows_per_w // CHUNK
    mesh = plsc.VectorSubcoreMesh(
        core_axis_name="c", subcore_axis_name="s",
        num_cores=SC_CORES, num_subcores=SC_SUBCORES)
    row_t = jax.ShapeDtypeStruct((n, dim), jnp.float32)

    @functools.partial(
        pl.kernel, mesh=mesh,
        compiler_params=pltpu.CompilerParams(use_tc_tiling_on_sc=False),
        out_type=[row_t, row_t, row_t],
        scratch_types=[
            pltpu.VMEM((nch, CHUNK), jnp.int32),
            pltpu.VMEM((nch, CHUNK), jnp.int32),
            pltpu.VMEM((2, CHUNK, dim), jnp.float32),
            pltpu.VMEM((2, CHUNK, dim), jnp.float32),
            pltpu.VMEM((2, CHUNK, dim), jnp.float32),
            pltpu.SemaphoreType.DMA,
            pltpu.SemaphoreType.DMA,
        ],
    )
    def body(q_hbm, k_hbm, v_hbm, pq_hbm, pk_hbm, qo_hbm, ko_hbm, vo_hbm,
             pq_v, pk_v, bq, bk, bv, sem_in, sem_out):
        wid = lax.axis_index("s") * SC_CORES + lax.axis_index("c")
        base = wid * rows_per_w
        pltpu.sync_copy(pq_hbm.at[wid], pq_v)
        pltpu.sync_copy(pk_hbm.at[wid], pk_v)

        def step(i, _):
            loads = []
            for b in range(2):
                c = i * 2 + b
                r0 = base + c * CHUNK
                loads.append(pltpu.async_copy(
                    q_hbm.at[pl.ds(r0, CHUNK)], bq.at[b], sem_in))
                loads.append(pltpu.async_copy(
                    k_hbm.at[pl.ds(r0, CHUNK)], bk.at[b], sem_in))
                loads.append(pltpu.async_copy(
                    v_hbm.at[pl.ds(r0, CHUNK)], bv.at[b], sem_in))
            for h in loads:
                h.wait()
            stores = []
            for b in range(2):
                c = i * 2 + b
                stores.append(pltpu.async_copy(
                    bq.at[b], qo_hbm.at[pq_v.at[c]], sem_out))
                stores.append(pltpu.async_copy(
                    bk.at[b], ko_hbm.at[pk_v.at[c]], sem_out))
                stores.append(pltpu.async_copy(
                    bv.at[b], vo_hbm.at[pk_v.at[c]], sem_out))
            for h in stores:
                h.wait()
            return 0

        lax.fori_loop(0, nch // 2, step, 0)

    return body(q2, k2, v2, posq3, posk3)


def _sc_gather_rows(src2, idx3, dim):
    n_out = idx3.shape[0] * idx3.shape[1] * idx3.shape[2]
    nch = idx3.shape[1]
    rows_per_w = nch * CHUNK
    mesh = plsc.VectorSubcoreMesh(
        core_axis_name="c", subcore_axis_name="s",
        num_cores=SC_CORES, num_subcores=SC_SUBCORES)

    @functools.partial(
        pl.kernel, mesh=mesh,
        compiler_params=pltpu.CompilerParams(use_tc_tiling_on_sc=False),
        out_type=jax.ShapeDtypeStruct((n_out, dim), jnp.float32),
        scratch_types=[
            pltpu.VMEM((nch, CHUNK), jnp.int32),
            pltpu.VMEM((2, CHUNK, dim), jnp.float32),
            pltpu.SemaphoreType.DMA,
            pltpu.SemaphoreType.DMA,
        ],
    )
    def body(src_hbm, idx_hbm, out_hbm, idx_v, buf, sem_in, sem_out):
        wid = lax.axis_index("s") * SC_CORES + lax.axis_index("c")
        base = wid * rows_per_w
        pltpu.sync_copy(idx_hbm.at[wid], idx_v)

        def step(i, _):
            loads = []
            for b in range(2):
                c = i * 2 + b
                loads.append(pltpu.async_copy(
                    src_hbm.at[idx_v.at[c]], buf.at[b], sem_in))
            for h in loads:
                h.wait()
            stores = []
            for b in range(2):
                c = i * 2 + b
                stores.append(pltpu.async_copy(
                    buf.at[b], out_hbm.at[pl.ds(base + c * CHUNK, CHUNK)],
                    sem_out))
            for h in stores:
                h.wait()
            return 0

        lax.fori_loop(0, nch // 2, step, 0)

    return body(src2, idx3)


def _sc_gather_two(a2, b2, idx3, dim):
    n_out = idx3.shape[0] * idx3.shape[1] * idx3.shape[2]
    nch = idx3.shape[1]
    rows_per_w = nch * CHUNK
    mesh = plsc.VectorSubcoreMesh(
        core_axis_name="c", subcore_axis_name="s",
        num_cores=SC_CORES, num_subcores=SC_SUBCORES)
    row_t = jax.ShapeDtypeStruct((n_out, dim), jnp.float32)

    @functools.partial(
        pl.kernel, mesh=mesh,
        compiler_params=pltpu.CompilerParams(use_tc_tiling_on_sc=False),
        out_type=[row_t, row_t],
        scratch_types=[
            pltpu.VMEM((nch, CHUNK), jnp.int32),
            pltpu.VMEM((2, CHUNK, dim), jnp.float32),
            pltpu.VMEM((2, CHUNK, dim), jnp.float32),
            pltpu.SemaphoreType.DMA,
            pltpu.SemaphoreType.DMA,
        ],
    )
    def body(a_hbm, b_hbm, idx_hbm, ao_hbm, bo_hbm, idx_v, bufa, bufb,
             sem_in, sem_out):
        wid = lax.axis_index("s") * SC_CORES + lax.axis_index("c")
        base = wid * rows_per_w
        pltpu.sync_copy(idx_hbm.at[wid], idx_v)
        for c in range(nch):
            b = c % 2
            ha = pltpu.async_copy(a_hbm.at[idx_v.at[c]], bufa.at[b], sem_in)
            hb = pltpu.async_copy(b_hbm.at[idx_v.at[c]], bufb.at[b], sem_in)
            ha.wait()
            hb.wait()
            r0 = base + c * CHUNK
            sa = pltpu.async_copy(bufa.at[b], ao_hbm.at[pl.ds(r0, CHUNK)],
                                  sem_out)
            sb = pltpu.async_copy(bufb.at[b], bo_hbm.at[pl.ds(r0, CHUNK)],
                                  sem_out)
            sa.wait()
            sb.wait()

    return body(a2, b2, idx3)


def _attn_body(qs_ref, ks_ref, vs_ref, ksub_ref, vsub_ref, cb_ref, o_ref, *,
               scale, log_w):
    blk = pl.program_id(1)
    fmin = float(np.finfo(np.float32).min)

    q = qs_ref[0]
    k = ks_ref[0]
    v = vs_ref[0]

    def dot_t(a, b):
        return lax.dot_general(a, b, (((1,), (1,)), ((), ())),
                               preferred_element_type=jnp.float32)

    def dot_n(a, b):
        return lax.dot_general(a, b, (((1,), (0,)), ((), ())),
                               preferred_element_type=jnp.float32)

    s1 = dot_t(q, k) * scale
    m1 = jnp.max(s1, axis=1, keepdims=True)
    l1 = m1 + jnp.log(jnp.sum(jnp.exp(s1 - m1), axis=1, keepdims=True))
    p1 = jnp.exp(s1 - l1)
    o1 = dot_n(p1, v)

    ksub = ksub_ref[0]
    vsub = vsub_ref[0]
    colb = cb_ref[0][0:1, :]
    bias = jnp.where(colb == jnp.float32(blk), fmin, 0.0)
    s2 = dot_t(q, ksub) * scale + bias
    m2 = jnp.max(s2, axis=1, keepdims=True)
    l2 = m2 + jnp.log(jnp.sum(jnp.exp(s2 - m2), axis=1, keepdims=True))
    p2 = jnp.exp(s2 - l2)
    o2 = dot_n(p2, vsub)
    l2 = l2 + log_w

    c = 1.0 / (1.0 + jnp.exp(l2 - l1))
    o_ref[0] = c * o1 + (1.0 - c) * o2


def _attention(qs3, ks3, vs3, ksub3, vsub3, colb3, bh, seq, dim):
    nblk = seq // BLOCK_SIZE
    scale = dim ** (-0.5)
    log_w = math.log(seq / SAMPLE_SIZE)
    return pl.pallas_call(
        functools.partial(_attn_body, scale=scale, log_w=log_w),
        grid=(bh, nblk),
        in_specs=[
            pl.BlockSpec((1, BLOCK_SIZE, dim), lambda i, j: (i, j, 0)),
            pl.BlockSpec((1, BLOCK_SIZE, dim), lambda i, j: (i, j, 0)),
            pl.BlockSpec((1, BLOCK_SIZE, dim), lambda i, j: (i, j, 0)),
            pl.BlockSpec((1, SAMPLE_SIZE, dim), lambda i, j: (i, 0, 0)),
            pl.BlockSpec((1, SAMPLE_SIZE, dim), lambda i, j: (i, 0, 0)),
            pl.BlockSpec((1, 8, SAMPLE_SIZE), lambda i, j: (i, 0, 0)),
        ],
        out_specs=pl.BlockSpec((1, BLOCK_SIZE, dim), lambda i, j: (i, j, 0)),
        out_shape=jax.ShapeDtypeStruct((bh, seq, dim), jnp.float32),
    )(qs3, ks3, vs3, ksub3, vsub3, colb3)


def kernel(query, key, value, proj_dir):
    B, H, S, D = query.shape
    bh = B * H
    n = bh * S

    w = proj_dir.reshape(D, LSH_NUM_PROJS)
    w_pad = jnp.concatenate([w, jnp.zeros((D, 1), w.dtype)], axis=1)

    q3 = query.reshape(bh, S, D)
    k3 = key.reshape(bh, S, D)
    v3 = value.reshape(bh, S, D)

    posq, posk = _hash_positions(q3, k3, w_pad, bh, S, D)
    posq3 = posq.reshape(SC_WORKERS, n // (SC_WORKERS * CHUNK), CHUNK)
    posk3 = posk.reshape(SC_WORKERS, n // (SC_WORKERS * CHUNK), CHUNK)

    q_sorted, k_sorted, v_sorted = _sc_sort_rows(
        q3.reshape(n, D), k3.reshape(n, D), v3.reshape(n, D),
        posq3, posk3, D)

    sampled = _sampled_set_np(B, H, S)
    samp_flat = (sampled.reshape(bh, SAMPLE_SIZE)
                 + (np.arange(bh, dtype=np.int32) * S)[:, None])
    m = bh * SAMPLE_SIZE
    samp_idx3 = jnp.asarray(
        samp_flat.reshape(SC_WORKERS, m // (SC_WORKERS * CHUNK), CHUNK))
    ksub, vsub = _sc_gather_two(k_sorted, v_sorted, samp_idx3, D)

    colb = np.broadcast_to(
        (sampled.reshape(bh, 1, SAMPLE_SIZE) // BLOCK_SIZE).astype(np.float32),
        (bh, 8, SAMPLE_SIZE)).copy()
    attn_sorted = _attention(
        q_sorted.reshape(bh, S, D), k_sorted.reshape(bh, S, D),
        v_sorted.reshape(bh, S, D),
        ksub.reshape(bh, SAMPLE_SIZE, D), vsub.reshape(bh, SAMPLE_SIZE, D),
        jnp.asarray(colb), bh, S, D)

    attn = _sc_gather_rows(attn_sorted.reshape(n, D), posq3, D)
    return attn.reshape(B, H, S, D)


# speedup vs baseline: 10.5582x; 1.3344x over previous
import functools
import math

import numpy as np
import jax
import jax.numpy as jnp
from jax import lax
from jax.experimental import pallas as pl
from jax.experimental.pallas import tpu as pltpu
from jax.experimental.pallas import tpu_sc as plsc

LSH_NUM_PROJS = 7
NUM_BUCKETS = 1 << LSH_NUM_PROJS
BLOCK_SIZE = 256
SAMPLE_SIZE = 256
DPAD = 128

SC_CORES = 2
SC_SUBCORES = 16
SC_WORKERS = SC_CORES * SC_SUBCORES
CHUNK = 128


def _unit_hamming_distance_array(size_n):
    a = np.array([0, 1], dtype=np.int32)
    for _ in range(size_n - 1):
        a = np.concatenate([a, np.flip(a) + a.shape[0]])
    return a


_PERM_NP = _unit_hamming_distance_array(LSH_NUM_PROJS).astype(np.int32)


@functools.lru_cache(maxsize=None)
def _sampled_set_np(b, h, n_key):
    with jax.ensure_compile_time_eval():
        s = jax.random.randint(jax.random.key(1234), (b, h, SAMPLE_SIZE), 0,
                               n_key)
        return np.asarray(jax.device_get(s)).astype(np.int32)


def _hash_pos_body(q_ref, k_ref, v_ref, w_ref, perm_ref,
                   posq_ref, posk_ref, qp_ref, kp_ref, vp_ref, *, seq, dim):
    bh = pl.program_id(0)
    w = w_ref[...]
    perm_row = perm_ref[...]
    lanes = lax.broadcasted_iota(jnp.int32, (1, NUM_BUCKETS), 1)
    rr = lax.broadcasted_iota(jnp.int32, (CHUNK, CHUNK), 0)
    cc = lax.broadcasted_iota(jnp.int32, (CHUNK, CHUNK), 1)
    tril_inc = (rr >= cc).astype(jnp.float32)
    lane8 = lax.broadcasted_iota(jnp.int32, (1, 8), 1)
    nch = seq // CHUNK

    def positions(x):
        proj = jnp.dot(x, w, preferred_element_type=jnp.float32)
        bits = (proj > 0).astype(jnp.int32)
        bin_id = jnp.sum(lax.shift_left(bits, lane8), axis=1,
                         keepdims=True)
        oh_bin = (bin_id == lanes).astype(jnp.int32)
        hsh = jnp.sum(oh_bin * perm_row, axis=1, keepdims=True)
        oh = (hsh == lanes).astype(jnp.float32)

        running = jnp.zeros((1, NUM_BUCKETS), jnp.float32)
        ranks = []
        for c in range(nch):
            blk = oh[c * CHUNK:(c + 1) * CHUNK, :]
            cum_c = (
                jax.lax.dot_general(
                    tril_inc, blk, (((1,), (0,)), ((), ())),
                    preferred_element_type=jnp.float32,
                )
                + running
            )
            ranks.append(jnp.sum(cum_c * blk, axis=1, keepdims=True))
            running = running + jnp.sum(blk, axis=0, keepdims=True)
        counts = running
        inc = counts
        for s in (1, 2, 4, 8, 16, 32, 64):
            shifted = jnp.concatenate(
                [jnp.zeros((1, s), jnp.float32), inc[:, :NUM_BUCKETS - s]],
                axis=1)
            inc = inc + shifted
        offs = inc - counts
        cols = []
        for c in range(nch):
            blk = oh[c * CHUNK:(c + 1) * CHUNK, :]
            off_c = jnp.sum(blk * offs, axis=1, keepdims=True)
            cols.append(off_c + ranks[c] - 1.0)
        pos_cols = jnp.concatenate(cols, axis=1)
        pos_mat = pos_cols.T
        return pos_mat.astype(jnp.int32) + bh * seq

    q = q_ref[0]
    k = k_ref[0]
    posq_ref[0] = positions(q)
    posk_ref[0] = positions(k)
    qp_ref[0, :, :dim] = q
    kp_ref[0, :, :dim] = k
    vp_ref[0, :, :dim] = v_ref[0]


def _hash_positions(q3, k3, v3, w_pad, bh, seq, dim):
    perm = jnp.asarray(_PERM_NP.reshape(1, NUM_BUCKETS))
    nch = seq // CHUNK
    pos_t = jax.ShapeDtypeStruct((bh, nch, CHUNK), jnp.int32)
    padded_t = jax.ShapeDtypeStruct((bh, seq, DPAD), jnp.float32)
    return pl.pallas_call(
        functools.partial(_hash_pos_body, seq=seq, dim=dim),
        grid=(bh,),
        in_specs=[
            pl.BlockSpec((1, seq, dim), lambda i: (i, 0, 0)),
            pl.BlockSpec((1, seq, dim), lambda i: (i, 0, 0)),
            pl.BlockSpec((1, seq, dim), lambda i: (i, 0, 0)),
            pl.BlockSpec((dim, 8), lambda i: (0, 0)),
            pl.BlockSpec((1, NUM_BUCKETS), lambda i: (0, 0)),
        ],
        out_specs=[
            pl.BlockSpec((1, nch, CHUNK), lambda i: (i, 0, 0)),
            pl.BlockSpec((1, nch, CHUNK), lambda i: (i, 0, 0)),
            pl.BlockSpec((1, seq, DPAD), lambda i: (i, 0, 0)),
            pl.BlockSpec((1, seq, DPAD), lambda i: (i, 0, 0)),
            pl.BlockSpec((1, seq, DPAD), lambda i: (i, 0, 0)),
        ],
        out_shape=[pos_t, pos_t, padded_t, padded_t, padded_t],
    )(q3, k3, v3, w_pad, perm)


def _sc_sort_rows(q2, k2, v2, posq3, posk3):
    n = q2.shape[0]
    rows_per_w = n // SC_WORKERS
    nch = rows_per_w // CHUNK
    mesh = plsc.VectorSubcoreMesh(
        core_axis_name="c", subcore_axis_name="s",
        num_cores=SC_CORES, num_subcores=SC_SUBCORES)
    row_t = jax.ShapeDtypeStruct((n, DPAD), jnp.float32)

    @functools.partial(
        pl.kernel, mesh=mesh,
        out_type=[row_t, row_t, row_t],
        scratch_types=[
            pltpu.VMEM((nch, CHUNK), jnp.int32),
            pltpu.VMEM((nch, CHUNK), jnp.int32),
            pltpu.VMEM((2, CHUNK, DPAD), jnp.float32),
            pltpu.VMEM((2, CHUNK, DPAD), jnp.float32),
            pltpu.VMEM((2, CHUNK, DPAD), jnp.float32),
            pltpu.SemaphoreType.DMA,
            pltpu.SemaphoreType.DMA,
        ],
    )
    def body(q_hbm, k_hbm, v_hbm, pq_hbm, pk_hbm, qo_hbm, ko_hbm, vo_hbm,
             pq_v, pk_v, bq, bk, bv, sem_in, sem_out):
        wid = lax.axis_index("s") * SC_CORES + lax.axis_index("c")
        base = wid * rows_per_w
        pltpu.sync_copy(pq_hbm.at[wid], pq_v)
        pltpu.sync_copy(pk_hbm.at[wid], pk_v)

        def step(i, _):
            loads = []
            for b in range(2):
                c = i * 2 + b
                r0 = base + c * CHUNK
                loads.append(pltpu.async_copy(
                    q_hbm.at[pl.ds(r0, CHUNK)], bq.at[b], sem_in))
                loads.append(pltpu.async_copy(
                    k_hbm.at[pl.ds(r0, CHUNK)], bk.at[b], sem_in))
                loads.append(pltpu.async_copy(
                    v_hbm.at[pl.ds(r0, CHUNK)], bv.at[b], sem_in))
            for h in loads:
                h.wait()
            stores = []
            for b in range(2):
                c = i * 2 + b
                stores.append(pltpu.async_copy(
                    bq.at[b], qo_hbm.at[pq_v.at[c]], sem_out))
                stores.append(pltpu.async_copy(
                    bk.at[b], ko_hbm.at[pk_v.at[c]], sem_out))
                stores.append(pltpu.async_copy(
                    bv.at[b], vo_hbm.at[pk_v.at[c]], sem_out))
            for h in stores:
                h.wait()
            return 0

        lax.fori_loop(0, nch // 2, step, 0)

    return body(q2, k2, v2, posq3, posk3)


def _sc_gather_rows(src2, idx3):
    n_out = idx3.shape[0] * idx3.shape[1] * idx3.shape[2]
    nch = idx3.shape[1]
    rows_per_w = nch * CHUNK
    mesh = plsc.VectorSubcoreMesh(
        core_axis_name="c", subcore_axis_name="s",
        num_cores=SC_CORES, num_subcores=SC_SUBCORES)

    @functools.partial(
        pl.kernel, mesh=mesh,
        out_type=jax.ShapeDtypeStruct((n_out, DPAD), jnp.float32),
        scratch_types=[
            pltpu.VMEM((nch, CHUNK), jnp.int32),
            pltpu.VMEM((2, CHUNK, DPAD), jnp.float32),
            pltpu.SemaphoreType.DMA,
            pltpu.SemaphoreType.DMA,
        ],
    )
    def body(src_hbm, idx_hbm, out_hbm, idx_v, buf, sem_in, sem_out):
        wid = lax.axis_index("s") * SC_CORES + lax.axis_index("c")
        base = wid * rows_per_w
        pltpu.sync_copy(idx_hbm.at[wid], idx_v)

        def step(i, _):
            loads = []
            for b in range(2):
                c = i * 2 + b
                loads.append(pltpu.async_copy(
                    src_hbm.at[idx_v.at[c]], buf.at[b], sem_in))
            for h in loads:
                h.wait()
            stores = []
            for b in range(2):
                c = i * 2 + b
                stores.append(pltpu.async_copy(
                    buf.at[b], out_hbm.at[pl.ds(base + c * CHUNK, CHUNK)],
                    sem_out))
            for h in stores:
                h.wait()
            return 0

        lax.fori_loop(0, nch // 2, step, 0)

    return body(src2, idx3)


def _sc_gather_two(a2, b2, idx3):
    n_out = idx3.shape[0] * idx3.shape[1] * idx3.shape[2]
    nch = idx3.shape[1]
    rows_per_w = nch * CHUNK
    mesh = plsc.VectorSubcoreMesh(
        core_axis_name="c", subcore_axis_name="s",
        num_cores=SC_CORES, num_subcores=SC_SUBCORES)
    row_t = jax.ShapeDtypeStruct((n_out, DPAD), jnp.float32)

    @functools.partial(
        pl.kernel, mesh=mesh,
        out_type=[row_t, row_t],
        scratch_types=[
            pltpu.VMEM((nch, CHUNK), jnp.int32),
            pltpu.VMEM((2, CHUNK, DPAD), jnp.float32),
            pltpu.VMEM((2, CHUNK, DPAD), jnp.float32),
            pltpu.SemaphoreType.DMA,
            pltpu.SemaphoreType.DMA,
        ],
    )
    def body(a_hbm, b_hbm, idx_hbm, ao_hbm, bo_hbm, idx_v, bufa, bufb,
             sem_in, sem_out):
        wid = lax.axis_index("s") * SC_CORES + lax.axis_index("c")
        base = wid * rows_per_w
        pltpu.sync_copy(idx_hbm.at[wid], idx_v)
        for c in range(nch):
            b = c % 2
            ha = pltpu.async_copy(a_hbm.at[idx_v.at[c]], bufa.at[b], sem_in)
            hb = pltpu.async_copy(b_hbm.at[idx_v.at[c]], bufb.at[b], sem_in)
            ha.wait()
            hb.wait()
            r0 = base + c * CHUNK
            sa = pltpu.async_copy(bufa.at[b], ao_hbm.at[pl.ds(r0, CHUNK)],
                                  sem_out)
            sb = pltpu.async_copy(bufb.at[b], bo_hbm.at[pl.ds(r0, CHUNK)],
                                  sem_out)
            sa.wait()
            sb.wait()

    return body(a2, b2, idx3)


def _attn_body(qs_ref, ks_ref, vs_ref, ksub_ref, vsub_ref, cb_ref, o_ref, *,
               scale, log_w, dim):
    blk = pl.program_id(1)
    fmin = float(np.finfo(np.float32).min)

    q = qs_ref[0][:, :dim]
    k = ks_ref[0][:, :dim]
    v = vs_ref[0][:, :dim]

    def dot_t(a, b):
        return lax.dot_general(a, b, (((1,), (1,)), ((), ())),
                               preferred_element_type=jnp.float32)

    def dot_n(a, b):
        return lax.dot_general(a, b, (((1,), (0,)), ((), ())),
                               preferred_element_type=jnp.float32)

    s1 = dot_t(q, k) * scale
    m1 = jnp.max(s1, axis=1, keepdims=True)
    l1 = m1 + jnp.log(jnp.sum(jnp.exp(s1 - m1), axis=1, keepdims=True))
    p1 = jnp.exp(s1 - l1)
    o1 = dot_n(p1, v)

    ksub = ksub_ref[0][:, :dim]
    vsub = vsub_ref[0][:, :dim]
    colb = cb_ref[0][0:1, :]
    bias = jnp.where(colb == jnp.float32(blk), fmin, 0.0)
    s2 = dot_t(q, ksub) * scale + bias
    m2 = jnp.max(s2, axis=1, keepdims=True)
    l2 = m2 + jnp.log(jnp.sum(jnp.exp(s2 - m2), axis=1, keepdims=True))
    p2 = jnp.exp(s2 - l2)
    o2 = dot_n(p2, vsub)
    l2 = l2 + log_w

    c = 1.0 / (1.0 + jnp.exp(l2 - l1))
    o_ref[0, :, :dim] = c * o1 + (1.0 - c) * o2


def _attention(qs3, ks3, vs3, ksub3, vsub3, colb3, bh, seq, dim):
    nblk = seq // BLOCK_SIZE
    scale = dim ** (-0.5)
    log_w = math.log(seq / SAMPLE_SIZE)
    return pl.pallas_call(
        functools.partial(_attn_body, scale=scale, log_w=log_w, dim=dim),
        grid=(bh, nblk),
        in_specs=[
            pl.BlockSpec((1, BLOCK_SIZE, DPAD), lambda i, j: (i, j, 0)),
            pl.BlockSpec((1, BLOCK_SIZE, DPAD), lambda i, j: (i, j, 0)),
            pl.BlockSpec((1, BLOCK_SIZE, DPAD), lambda i, j: (i, j, 0)),
            pl.BlockSpec((1, SAMPLE_SIZE, DPAD), lambda i, j: (i, 0, 0)),
            pl.BlockSpec((1, SAMPLE_SIZE, DPAD), lambda i, j: (i, 0, 0)),
            pl.BlockSpec((1, 8, SAMPLE_SIZE), lambda i, j: (i, 0, 0)),
        ],
        out_specs=pl.BlockSpec((1, BLOCK_SIZE, DPAD), lambda i, j: (i, j, 0)),
        out_shape=jax.ShapeDtypeStruct((bh, seq, DPAD), jnp.float32),
    )(qs3, ks3, vs3, ksub3, vsub3, colb3)


def kernel(query, key, value, proj_dir):
    B, H, S, D = query.shape
    bh = B * H
    n = bh * S

    w = proj_dir.reshape(D, LSH_NUM_PROJS)
    w_pad = jnp.concatenate([w, jnp.zeros((D, 1), w.dtype)], axis=1)

    q3 = query.reshape(bh, S, D)
    k3 = key.reshape(bh, S, D)
    v3 = value.reshape(bh, S, D)

    posq, posk, qp, kp, vp = _hash_positions(q3, k3, v3, w_pad, bh, S, D)

    q_sorted, k_sorted, v_sorted = _sc_sort_rows(
        qp.reshape(n, DPAD), kp.reshape(n, DPAD), vp.reshape(n, DPAD),
        posq, posk)

    sampled = _sampled_set_np(B, H, S)
    samp_flat = (sampled.reshape(bh, SAMPLE_SIZE)
                 + (np.arange(bh, dtype=np.int32) * S)[:, None])
    m = bh * SAMPLE_SIZE
    samp_idx3 = jnp.asarray(
        samp_flat.reshape(SC_WORKERS, m // (SC_WORKERS * CHUNK), CHUNK))
    ksub, vsub = _sc_gather_two(k_sorted, v_sorted, samp_idx3)

    colb = np.broadcast_to(
        (sampled.reshape(bh, 1, SAMPLE_SIZE) // BLOCK_SIZE).astype(np.float32),
        (bh, 8, SAMPLE_SIZE)).copy()
    attn_sorted = _attention(
        q_sorted.reshape(bh, S, DPAD), k_sorted.reshape(bh, S, DPAD),
        v_sorted.reshape(bh, S, DPAD),
        ksub.reshape(bh, SAMPLE_SIZE, DPAD), vsub.reshape(bh, SAMPLE_SIZE, DPAD),
        jnp.asarray(colb), bh, S, D)

    attn = _sc_gather_rows(attn_sorted.reshape(n, DPAD), posq)
    return attn.reshape(B, H, S, DPAD)[..., :D]
